# Initial kernel scaffold; baseline (speedup 1.0000x reference)
import jax
import jax.numpy as jnp
from jax.experimental import pallas as pl


def kernel(x, edge_attr, params, edge_index):
    raise NotImplementedError("write your pallas kernel here")


# speedup vs baseline: 1.5105x; 1.5105x over previous
import functools

import jax
import jax.numpy as jnp
from jax import lax
from jax.experimental import pallas as pl
from jax.experimental.pallas import tpu as pltpu
from jax.experimental.pallas import tpu_sc as plsc

N = 10000
E = 160000
LATENT = 64
STEPS = 3

NC = 2
NS = 16
NW = NC * NS
CH = 128
E_PAD = ((E + NW * CH - 1) // (NW * CH)) * (NW * CH)
BLK_E = 2048
GRID_E = E_PAD // BLK_E
BLK_N = 1000
GRID_N = N // BLK_N
ACC_ROWS = ((N + NS - 1) // NS + 1) * NS
ROWS_PER_TILE = ACC_ROWS // NS


def _ln(h, g, beta):
    mu = jnp.mean(h, axis=-1, keepdims=True)
    var = jnp.mean((h - mu) ** 2, axis=-1, keepdims=True)
    return (h - mu) * jax.lax.rsqrt(var + 1e-5) * g + beta


def _relu(v):
    return jnp.maximum(v, 0.0)


def _enc_body(x_ref, w1_ref, b1_ref, w2_ref, b2_ref, g_ref, beta_ref, o_ref):
    h = _relu(jnp.dot(x_ref[...], w1_ref[...],
                      preferred_element_type=jnp.float32) + b1_ref[...])
    h = _relu(jnp.dot(h, w2_ref[...],
                      preferred_element_type=jnp.float32) + b2_ref[...])
    o_ref[...] = _ln(h, g_ref[...], beta_ref[...])


def _edge_body(a_ref, b_ref, ne_ref, w1a_ref, w1b_ref, w1c_ref, b1_ref,
               w2_ref, b2_ref, g_ref, beta_ref, msg_ref, neo_ref):
    a = a_ref[...]
    b = b_ref[...]
    ne = ne_ref[...]
    w1a = w1a_ref[...]
    w1b = w1b_ref[...]
    pc = jnp.dot(ne, w1c_ref[...], preferred_element_type=jnp.float32) + b1_ref[...]
    u = jnp.dot(a, w1a, preferred_element_type=jnp.float32) + \
        jnp.dot(b, w1b, preferred_element_type=jnp.float32) + pc
    v = jnp.dot(b, w1a, preferred_element_type=jnp.float32) + \
        jnp.dot(a, w1b, preferred_element_type=jnp.float32) + pc
    w2 = w2_ref[...]
    b2 = b2_ref[...]
    g = g_ref[...]
    beta = beta_ref[...]
    hu = _relu(jnp.dot(_relu(u), w2, preferred_element_type=jnp.float32) + b2)
    hv = _relu(jnp.dot(_relu(v), w2, preferred_element_type=jnp.float32) + b2)
    msg_ref[...] = _ln(hu, g, beta)
    neo_ref[...] = ne + _ln(hv, g, beta)


def _node_body(p0_ref, p1_ref, nx_ref, w1a_ref, w1b_ref, b1_ref,
               w2_ref, b2_ref, g_ref, beta_ref, nxo_ref):
    aggr = p0_ref[0] + p1_ref[0]
    nx = nx_ref[...]
    u = jnp.dot(aggr, w1a_ref[...], preferred_element_type=jnp.float32) + \
        jnp.dot(nx, w1b_ref[...], preferred_element_type=jnp.float32) + b1_ref[...]
    h = _relu(jnp.dot(_relu(u), w2_ref[...],
                      preferred_element_type=jnp.float32) + b2_ref[...])
    nxo_ref[...] = nx + _ln(h, g_ref[...], beta_ref[...])


def _gru(xv, wr, wz, wn, cr, cz, cn, hn):
    r = jax.nn.sigmoid(jnp.dot(xv, wr, preferred_element_type=jnp.float32) + cr)
    z = jax.nn.sigmoid(jnp.dot(xv, wz, preferred_element_type=jnp.float32) + cz)
    n = jnp.tanh(jnp.dot(xv, wn, preferred_element_type=jnp.float32) + cn + r * hn)
    return (1.0 - z) * n


def _decode_body(nx_ref, s_ref,
                 wr1_ref, wz1_ref, wn1_ref, cr1_ref, cz1_ref, cn1_ref, hn1_ref,
                 wr2_ref, wz2_ref, wn2_ref, cr2_ref, cz2_ref, cn2_ref, hn2_ref,
                 dw1a_ref, dw1b_ref, dw1c_ref, db1_ref, w2p_ref, b2p_ref,
                 o_ref):
    nx = nx_ref[...]
    h1 = _gru(nx, wr1_ref[...], wz1_ref[...], wn1_ref[...],
              cr1_ref[...], cz1_ref[...], cn1_ref[...], hn1_ref[...])
    h2 = _gru(h1, wr2_ref[...], wz2_ref[...], wn2_ref[...],
              cr2_ref[...], cz2_ref[...], cn2_ref[...], hn2_ref[...])
    hh = _relu(jnp.dot(h1, dw1a_ref[...], preferred_element_type=jnp.float32) +
               jnp.dot(h2, dw1b_ref[...], preferred_element_type=jnp.float32) +
               jnp.dot(s_ref[...], dw1c_ref[...], preferred_element_type=jnp.float32) +
               db1_ref[...])
    o_ref[...] = jnp.dot(hh, w2p_ref[...],
                         preferred_element_type=jnp.float32) + b2p_ref[...]


def _full_spec(shape):
    return pl.BlockSpec(shape, lambda i: (0,) * len(shape))


def _row_spec(blk, width):
    return pl.BlockSpec((blk, width), lambda i: (i, 0))


def _run_enc(xp, p, rows, blk, fin):
    grid = rows // blk
    return pl.pallas_call(
        _enc_body,
        grid=(grid,),
        in_specs=[
            _row_spec(blk, fin),
            _full_spec((fin, LATENT)), _full_spec((1, LATENT)),
            _full_spec((LATENT, LATENT)), _full_spec((1, LATENT)),
            _full_spec((1, LATENT)), _full_spec((1, LATENT)),
        ],
        out_specs=_row_spec(blk, LATENT),
        out_shape=jax.ShapeDtypeStruct((rows, LATENT), jnp.float32),
    )(xp, p["W1"], p["b1"].reshape(1, -1), p["W2"], p["b2"].reshape(1, -1),
      p["g"].reshape(1, -1), p["beta"].reshape(1, -1))


def _run_edge(gathered, ne, p):
    w1a = p["W1"][:LATENT]
    w1b = p["W1"][LATENT:2 * LATENT]
    w1c = p["W1"][2 * LATENT:]
    nblk_a = E_PAD // BLK_E
    return pl.pallas_call(
        _edge_body,
        grid=(GRID_E,),
        in_specs=[
            pl.BlockSpec((BLK_E, LATENT), lambda i: (i, 0)),
            pl.BlockSpec((BLK_E, LATENT), lambda i, n=nblk_a: (n + i, 0)),
            _row_spec(BLK_E, LATENT),
            _full_spec((LATENT, LATENT)), _full_spec((LATENT, LATENT)),
            _full_spec((LATENT, LATENT)), _full_spec((1, LATENT)),
            _full_spec((LATENT, LATENT)), _full_spec((1, LATENT)),
            _full_spec((1, LATENT)), _full_spec((1, LATENT)),
        ],
        out_specs=[_row_spec(BLK_E, LATENT), _row_spec(BLK_E, LATENT)],
        out_shape=[jax.ShapeDtypeStruct((E_PAD, LATENT), jnp.float32),
                   jax.ShapeDtypeStruct((E_PAD, LATENT), jnp.float32)],
    )(gathered, gathered, ne, w1a, w1b, w1c, p["b1"].reshape(1, -1),
      p["W2"], p["b2"].reshape(1, -1), p["g"].reshape(1, -1),
      p["beta"].reshape(1, -1))


def _run_node(partials, nx, p):
    w1a = p["W1"][:LATENT]
    w1b = p["W1"][LATENT:]
    return pl.pallas_call(
        _node_body,
        grid=(GRID_N,),
        in_specs=[
            pl.BlockSpec((1, BLK_N, LATENT), lambda i: (0, i, 0)),
            pl.BlockSpec((1, BLK_N, LATENT), lambda i: (1, i, 0)),
            _row_spec(BLK_N, LATENT),
            _full_spec((LATENT, LATENT)), _full_spec((LATENT, LATENT)),
            _full_spec((1, LATENT)),
            _full_spec((LATENT, LATENT)), _full_spec((1, LATENT)),
            _full_spec((1, LATENT)), _full_spec((1, LATENT)),
        ],
        out_specs=_row_spec(BLK_N, LATENT),
        out_shape=jax.ShapeDtypeStruct((N, LATENT), jnp.float32),
    )(partials, partials, nx, w1a, w1b, p["b1"].reshape(1, -1),
      p["W2"], p["b2"].reshape(1, -1), p["g"].reshape(1, -1),
      p["beta"].reshape(1, -1))


def _gru_args(p):
    wr = p["Wih"][:, :LATENT]
    wz = p["Wih"][:, LATENT:2 * LATENT]
    wn = p["Wih"][:, 2 * LATENT:]
    cr = (p["bih"][:LATENT] + p["bhh"][:LATENT]).reshape(1, -1)
    cz = (p["bih"][LATENT:2 * LATENT] + p["bhh"][LATENT:2 * LATENT]).reshape(1, -1)
    cn = p["bih"][2 * LATENT:].reshape(1, -1)
    hn = p["bhh"][2 * LATENT:].reshape(1, -1)
    return wr, wz, wn, cr, cz, cn, hn


def _run_decode(nx, s, params):
    g1 = _gru_args(params["gru1"])
    g2 = _gru_args(params["gru2"])
    dec = params["dec"]
    dw1a = dec["W1"][:LATENT]
    dw1b = dec["W1"][LATENT:2 * LATENT]
    dw1c = dec["W1"][2 * LATENT:]
    out_dim = dec["W2"].shape[1]
    w2p = jnp.zeros((LATENT, 128), jnp.float32).at[:, :out_dim].set(dec["W2"])
    b2p = jnp.zeros((1, 128), jnp.float32).at[:, :out_dim].set(dec["b2"])
    gru_specs = [_full_spec((LATENT, LATENT))] * 3 + [_full_spec((1, LATENT))] * 4
    out_pad = pl.pallas_call(
        _decode_body,
        grid=(GRID_N,),
        in_specs=[_row_spec(BLK_N, LATENT), _row_spec(BLK_N, LATENT)]
                 + gru_specs + gru_specs
                 + [_full_spec((LATENT, LATENT))] * 3
                 + [_full_spec((1, LATENT)),
                    _full_spec((LATENT, 128)), _full_spec((1, 128))],
        out_specs=_row_spec(BLK_N, 128),
        out_shape=jax.ShapeDtypeStruct((N, 128), jnp.float32),
    )(nx, s, *g1, *g2, dw1a, dw1b, dw1c, dec["b1"].reshape(1, -1), w2p, b2p)
    return out_pad[:, :out_dim]


_G_PER_W = 2 * E_PAD // NW
_G_NCH = _G_PER_W // CH
_S_PER_W = E_PAD // NW
_S_NCH = _S_PER_W // CH


@functools.cache
def _sc_kernels():
    mesh = plsc.VectorSubcoreMesh(core_axis_name="c", subcore_axis_name="s",
                                  num_cores=NC, num_subcores=NS)

    @functools.partial(
        pl.kernel,
        out_type=jax.ShapeDtypeStruct((2 * E_PAD, LATENT), jnp.float32),
        mesh=mesh,
        scratch_types=[
            pltpu.VMEM((_G_NCH, CH), jnp.int32),
            pltpu.VMEM((CH, LATENT), jnp.float32),
            pltpu.SemaphoreType.DMA,
        ],
        compiler_params=pltpu.CompilerParams(use_tc_tiling_on_sc=False),
    )
    def gather_k(table_hbm, idx_hbm, out_hbm, idx_v, rows_v, sem):
        wid = lax.axis_index("s") * NC + lax.axis_index("c")
        pltpu.sync_copy(idx_hbm.at[wid], idx_v)
        base = wid * _G_PER_W

        @pl.loop(0, _G_NCH)
        def _chunk(j):
            pltpu.async_copy(table_hbm.at[idx_v.at[j]], rows_v, sem).wait()
            pltpu.sync_copy(rows_v, out_hbm.at[pl.ds(base + j * CH, CH)])

    @functools.partial(
        pl.kernel,
        out_type=jax.ShapeDtypeStruct((NC, ACC_ROWS, LATENT), jnp.float32),
        mesh=mesh,
        scratch_types=[
            pltpu.VMEM((_S_NCH, CH), jnp.int32),
            pltpu.VMEM((CH, LATENT), jnp.float32),
            pltpu.VMEM_SHARED((ACC_ROWS, LATENT), jnp.float32),
        ],
        compiler_params=pltpu.CompilerParams(use_tc_tiling_on_sc=False),
    )
    def scatter_k(msg_hbm, idx_hbm, zeros_hbm, out_hbm, idx_v, msg_v, acc_sh):
        cid = lax.axis_index("c")
        sid = lax.axis_index("s")
        wid = sid * NC + cid
        pltpu.sync_copy(zeros_hbm.at[pl.ds(sid * ROWS_PER_TILE, ROWS_PER_TILE)],
                        acc_sh.at[pl.ds(sid * ROWS_PER_TILE, ROWS_PER_TILE)])
        pltpu.sync_copy(idx_hbm.at[wid], idx_v)
        plsc.subcore_barrier()
        base = wid * _S_PER_W

        @pl.loop(0, _S_NCH)
        def _chunk(j):
            pltpu.sync_copy(msg_hbm.at[pl.ds(base + j * CH, CH)], msg_v)
            pltpu.sync_copy(msg_v, acc_sh.at[idx_v.at[j]], add=True)

        plsc.subcore_barrier()
        pltpu.sync_copy(acc_sh.at[pl.ds(sid * ROWS_PER_TILE, ROWS_PER_TILE)],
                        out_hbm.at[cid, pl.ds(sid * ROWS_PER_TILE, ROWS_PER_TILE)])

    return gather_k, scatter_k


def _sc_gather(table, gidx):
    return _sc_kernels()[0](table, gidx)


def _sc_scatter(msg, sidx, zeros_acc):
    return _sc_kernels()[1](msg, sidx, zeros_acc)


def kernel(x, edge_attr, params, edge_index):
    row = edge_index[0].astype(jnp.int32)
    col = edge_index[1].astype(jnp.int32)
    pad = E_PAD - E
    zero_idx = jnp.zeros((pad,), jnp.int32)
    gidx = jnp.concatenate([col, zero_idx, row, zero_idx]).reshape(NW, _G_NCH, CH)
    sidx = jnp.concatenate([col, jnp.full((pad,), N, jnp.int32)]).reshape(
        NW, _S_NCH, CH)
    zeros_acc = jnp.zeros((ACC_ROWS, LATENT), jnp.float32)

    node_lat = _run_enc(x, params["node_enc"], N, BLK_N, x.shape[1])
    ea_pad = jnp.pad(edge_attr, ((0, pad), (0, 0)))
    edge_lat = _run_enc(ea_pad, params["edge_enc"], E_PAD, BLK_E,
                        edge_attr.shape[1])

    nx = node_lat
    ne = edge_lat
    for _ in range(STEPS):
        gathered = _sc_gather(nx, gidx)
        msg, ne = _run_edge(gathered, ne, params["edge_net"])
        partials = _sc_scatter(msg, sidx, zeros_acc)
        nx = _run_node(partials, nx, params["node_net"])

    return _run_decode(nx, node_lat, params)


# speedup vs baseline: 1.6431x; 1.0878x over previous
import functools

import jax
import jax.numpy as jnp
from jax import lax
from jax.experimental import pallas as pl
from jax.experimental.pallas import tpu as pltpu
from jax.experimental.pallas import tpu_sc as plsc

N = 10000
E = 160000
LATENT = 64
STEPS = 3

NC = 2
NS = 16
NW = NC * NS
CH = 128
E_PAD = ((E + NW * CH - 1) // (NW * CH)) * (NW * CH)
BLK_E = 2048
GRID_E = E_PAD // BLK_E
BLK_N = 1000
GRID_N = N // BLK_N
ACC_ROWS = ((N + NS - 1) // NS + 1) * NS
ROWS_PER_TILE = ACC_ROWS // NS


def _ln(h, g, beta):
    mu = jnp.mean(h, axis=-1, keepdims=True)
    var = jnp.mean((h - mu) ** 2, axis=-1, keepdims=True)
    return (h - mu) * jax.lax.rsqrt(var + 1e-5) * g + beta


def _relu(v):
    return jnp.maximum(v, 0.0)


def _enc_body(x_ref, w1_ref, b1_ref, w2_ref, b2_ref, g_ref, beta_ref, o_ref):
    h = _relu(jnp.dot(x_ref[...], w1_ref[...],
                      preferred_element_type=jnp.float32) + b1_ref[...])
    h = _relu(jnp.dot(h, w2_ref[...],
                      preferred_element_type=jnp.float32) + b2_ref[...])
    o_ref[...] = _ln(h, g_ref[...], beta_ref[...])


def _edge_body(a_ref, b_ref, ne_ref, w1a_ref, w1b_ref, w1c_ref, b1_ref,
               w2_ref, b2_ref, g_ref, beta_ref, msg_ref, neo_ref):
    a = a_ref[...]
    b = b_ref[...]
    ne = ne_ref[...]
    w1a = w1a_ref[...]
    w1b = w1b_ref[...]
    pc = jnp.dot(ne, w1c_ref[...], preferred_element_type=jnp.float32) + b1_ref[...]
    u = jnp.dot(a, w1a, preferred_element_type=jnp.float32) + \
        jnp.dot(b, w1b, preferred_element_type=jnp.float32) + pc
    v = jnp.dot(b, w1a, preferred_element_type=jnp.float32) + \
        jnp.dot(a, w1b, preferred_element_type=jnp.float32) + pc
    w2 = w2_ref[...]
    b2 = b2_ref[...]
    g = g_ref[...]
    beta = beta_ref[...]
    hu = _relu(jnp.dot(_relu(u), w2, preferred_element_type=jnp.float32) + b2)
    hv = _relu(jnp.dot(_relu(v), w2, preferred_element_type=jnp.float32) + b2)
    msg_ref[...] = _ln(hu, g, beta)
    neo_ref[...] = ne + _ln(hv, g, beta)


def _node_body(p0_ref, p1_ref, nx_ref, w1a_ref, w1b_ref, b1_ref,
               w2_ref, b2_ref, g_ref, beta_ref, nxo_ref):
    aggr = p0_ref[0] + p1_ref[0]
    nx = nx_ref[...]
    u = jnp.dot(aggr, w1a_ref[...], preferred_element_type=jnp.float32) + \
        jnp.dot(nx, w1b_ref[...], preferred_element_type=jnp.float32) + b1_ref[...]
    h = _relu(jnp.dot(_relu(u), w2_ref[...],
                      preferred_element_type=jnp.float32) + b2_ref[...])
    nxo_ref[...] = nx + _ln(h, g_ref[...], beta_ref[...])


def _gru(xv, wr, wz, wn, cr, cz, cn, hn):
    r = jax.nn.sigmoid(jnp.dot(xv, wr, preferred_element_type=jnp.float32) + cr)
    z = jax.nn.sigmoid(jnp.dot(xv, wz, preferred_element_type=jnp.float32) + cz)
    n = jnp.tanh(jnp.dot(xv, wn, preferred_element_type=jnp.float32) + cn + r * hn)
    return (1.0 - z) * n


def _decode_body(nx_ref, s_ref,
                 wr1_ref, wz1_ref, wn1_ref, cr1_ref, cz1_ref, cn1_ref, hn1_ref,
                 wr2_ref, wz2_ref, wn2_ref, cr2_ref, cz2_ref, cn2_ref, hn2_ref,
                 dw1a_ref, dw1b_ref, dw1c_ref, db1_ref, w2p_ref, b2p_ref,
                 o_ref):
    nx = nx_ref[...]
    h1 = _gru(nx, wr1_ref[...], wz1_ref[...], wn1_ref[...],
              cr1_ref[...], cz1_ref[...], cn1_ref[...], hn1_ref[...])
    h2 = _gru(h1, wr2_ref[...], wz2_ref[...], wn2_ref[...],
              cr2_ref[...], cz2_ref[...], cn2_ref[...], hn2_ref[...])
    hh = _relu(jnp.dot(h1, dw1a_ref[...], preferred_element_type=jnp.float32) +
               jnp.dot(h2, dw1b_ref[...], preferred_element_type=jnp.float32) +
               jnp.dot(s_ref[...], dw1c_ref[...], preferred_element_type=jnp.float32) +
               db1_ref[...])
    o_ref[...] = jnp.dot(hh, w2p_ref[...],
                         preferred_element_type=jnp.float32) + b2p_ref[...]


def _full_spec(shape):
    return pl.BlockSpec(shape, lambda i: (0,) * len(shape))


def _row_spec(blk, width):
    return pl.BlockSpec((blk, width), lambda i: (i, 0))


def _run_enc(xp, p, rows, blk, fin):
    grid = rows // blk
    return pl.pallas_call(
        _enc_body,
        grid=(grid,),
        in_specs=[
            _row_spec(blk, fin),
            _full_spec((fin, LATENT)), _full_spec((1, LATENT)),
            _full_spec((LATENT, LATENT)), _full_spec((1, LATENT)),
            _full_spec((1, LATENT)), _full_spec((1, LATENT)),
        ],
        out_specs=_row_spec(blk, LATENT),
        out_shape=jax.ShapeDtypeStruct((rows, LATENT), jnp.float32),
    )(xp, p["W1"], p["b1"].reshape(1, -1), p["W2"], p["b2"].reshape(1, -1),
      p["g"].reshape(1, -1), p["beta"].reshape(1, -1))


def _run_edge(gathered, ne, p):
    w1a = p["W1"][:LATENT]
    w1b = p["W1"][LATENT:2 * LATENT]
    w1c = p["W1"][2 * LATENT:]
    nblk_a = E_PAD // BLK_E
    return pl.pallas_call(
        _edge_body,
        grid=(GRID_E,),
        in_specs=[
            pl.BlockSpec((BLK_E, LATENT), lambda i: (i, 0)),
            pl.BlockSpec((BLK_E, LATENT), lambda i, n=nblk_a: (n + i, 0)),
            _row_spec(BLK_E, LATENT),
            _full_spec((LATENT, LATENT)), _full_spec((LATENT, LATENT)),
            _full_spec((LATENT, LATENT)), _full_spec((1, LATENT)),
            _full_spec((LATENT, LATENT)), _full_spec((1, LATENT)),
            _full_spec((1, LATENT)), _full_spec((1, LATENT)),
        ],
        out_specs=[_row_spec(BLK_E, LATENT), _row_spec(BLK_E, LATENT)],
        out_shape=[jax.ShapeDtypeStruct((E_PAD, LATENT), jnp.float32),
                   jax.ShapeDtypeStruct((E_PAD, LATENT), jnp.float32)],
    )(gathered, gathered, ne, w1a, w1b, w1c, p["b1"].reshape(1, -1),
      p["W2"], p["b2"].reshape(1, -1), p["g"].reshape(1, -1),
      p["beta"].reshape(1, -1))


def _run_node(partials, nx, p):
    w1a = p["W1"][:LATENT]
    w1b = p["W1"][LATENT:]
    return pl.pallas_call(
        _node_body,
        grid=(GRID_N,),
        in_specs=[
            pl.BlockSpec((1, BLK_N, LATENT), lambda i: (0, i, 0)),
            pl.BlockSpec((1, BLK_N, LATENT), lambda i: (1, i, 0)),
            _row_spec(BLK_N, LATENT),
            _full_spec((LATENT, LATENT)), _full_spec((LATENT, LATENT)),
            _full_spec((1, LATENT)),
            _full_spec((LATENT, LATENT)), _full_spec((1, LATENT)),
            _full_spec((1, LATENT)), _full_spec((1, LATENT)),
        ],
        out_specs=_row_spec(BLK_N, LATENT),
        out_shape=jax.ShapeDtypeStruct((N, LATENT), jnp.float32),
    )(partials, partials, nx, w1a, w1b, p["b1"].reshape(1, -1),
      p["W2"], p["b2"].reshape(1, -1), p["g"].reshape(1, -1),
      p["beta"].reshape(1, -1))


def _gru_args(p):
    wr = p["Wih"][:, :LATENT]
    wz = p["Wih"][:, LATENT:2 * LATENT]
    wn = p["Wih"][:, 2 * LATENT:]
    cr = (p["bih"][:LATENT] + p["bhh"][:LATENT]).reshape(1, -1)
    cz = (p["bih"][LATENT:2 * LATENT] + p["bhh"][LATENT:2 * LATENT]).reshape(1, -1)
    cn = p["bih"][2 * LATENT:].reshape(1, -1)
    hn = p["bhh"][2 * LATENT:].reshape(1, -1)
    return wr, wz, wn, cr, cz, cn, hn


def _run_decode(nx, s, params):
    g1 = _gru_args(params["gru1"])
    g2 = _gru_args(params["gru2"])
    dec = params["dec"]
    dw1a = dec["W1"][:LATENT]
    dw1b = dec["W1"][LATENT:2 * LATENT]
    dw1c = dec["W1"][2 * LATENT:]
    out_dim = dec["W2"].shape[1]
    w2p = jnp.zeros((LATENT, 128), jnp.float32).at[:, :out_dim].set(dec["W2"])
    b2p = jnp.zeros((1, 128), jnp.float32).at[:, :out_dim].set(dec["b2"])
    gru_specs = [_full_spec((LATENT, LATENT))] * 3 + [_full_spec((1, LATENT))] * 4
    out_pad = pl.pallas_call(
        _decode_body,
        grid=(GRID_N,),
        in_specs=[_row_spec(BLK_N, LATENT), _row_spec(BLK_N, LATENT)]
                 + gru_specs + gru_specs
                 + [_full_spec((LATENT, LATENT))] * 3
                 + [_full_spec((1, LATENT)),
                    _full_spec((LATENT, 128)), _full_spec((1, 128))],
        out_specs=_row_spec(BLK_N, 128),
        out_shape=jax.ShapeDtypeStruct((N, 128), jnp.float32),
    )(nx, s, *g1, *g2, dw1a, dw1b, dw1c, dec["b1"].reshape(1, -1), w2p, b2p)
    return out_pad[:, :out_dim]


_G_PER_W = 2 * E_PAD // NW
_G_NCH = _G_PER_W // CH
_S_PER_W = E_PAD // NW
_S_NCH = _S_PER_W // CH


_G_GRP = 4
_G_NGRP = _G_NCH // _G_GRP
_G_ROWS = _G_GRP * CH
_S_GRP = 4
_S_NGRP = _S_NCH // _S_GRP
_S_ROWS = _S_GRP * CH


@functools.cache
def _sc_kernels():
    mesh = plsc.VectorSubcoreMesh(core_axis_name="c", subcore_axis_name="s",
                                  num_cores=NC, num_subcores=NS)

    @functools.partial(
        pl.kernel,
        out_type=jax.ShapeDtypeStruct((2 * E_PAD, LATENT), jnp.float32),
        mesh=mesh,
        scratch_types=[
            pltpu.VMEM((_G_NCH, CH), jnp.int32),
            pltpu.VMEM((_G_ROWS, LATENT), jnp.float32),
            pltpu.VMEM((_G_ROWS, LATENT), jnp.float32),
            pltpu.SemaphoreType.DMA((2,)),
            pltpu.SemaphoreType.DMA((2,)),
        ],
        compiler_params=pltpu.CompilerParams(use_tc_tiling_on_sc=False),
    )
    def gather_k(table_hbm, idx_hbm, out_hbm, idx_v, rows0, rows1, gsem, wsem):
        wid = lax.axis_index("s") * NC + lax.axis_index("c")
        pltpu.sync_copy(idx_hbm.at[wid], idx_v)
        base = wid * _G_PER_W
        rows = (rows0, rows1)

        def fire(g, buf):
            for q in range(_G_GRP):
                pltpu.async_copy(table_hbm.at[idx_v.at[g * _G_GRP + q]],
                                 rows[buf].at[pl.ds(q * CH, CH)], gsem.at[buf])

        fire(0, 0)

        @pl.loop(0, _G_NGRP, step=2)
        def _grp(g0):
            for p in range(2):
                g = g0 + p
                pltpu.make_async_copy(table_hbm.at[pl.ds(0, _G_ROWS)],
                                      rows[p], gsem.at[p]).wait()

                @pl.when(g >= 1)
                def _():
                    pltpu.make_async_copy(rows[1 - p],
                                          out_hbm.at[pl.ds(0, _G_ROWS)],
                                          wsem.at[1 - p]).wait()

                @pl.when(g + 1 < _G_NGRP)
                def _():
                    fire(g + 1, 1 - p)

                pltpu.async_copy(rows[p],
                                 out_hbm.at[pl.ds(base + g * _G_ROWS, _G_ROWS)],
                                 wsem.at[p])

        pltpu.make_async_copy(rows[1], out_hbm.at[pl.ds(0, _G_ROWS)],
                              wsem.at[1]).wait()

    @functools.partial(
        pl.kernel,
        out_type=jax.ShapeDtypeStruct((NC, ACC_ROWS, LATENT), jnp.float32),
        mesh=mesh,
        scratch_types=[
            pltpu.VMEM((_S_NCH, CH), jnp.int32),
            pltpu.VMEM((_S_ROWS, LATENT), jnp.float32),
            pltpu.VMEM((_S_ROWS, LATENT), jnp.float32),
            pltpu.VMEM_SHARED((ACC_ROWS, LATENT), jnp.float32),
            pltpu.SemaphoreType.DMA((2,)),
        ],
        compiler_params=pltpu.CompilerParams(use_tc_tiling_on_sc=False),
    )
    def scatter_k(msg_hbm, idx_hbm, zeros_hbm, out_hbm, idx_v, msg0, msg1,
                  acc_sh, lsem):
        cid = lax.axis_index("c")
        sid = lax.axis_index("s")
        wid = sid * NC + cid
        pltpu.sync_copy(zeros_hbm.at[pl.ds(sid * ROWS_PER_TILE, ROWS_PER_TILE)],
                        acc_sh.at[pl.ds(sid * ROWS_PER_TILE, ROWS_PER_TILE)])
        pltpu.sync_copy(idx_hbm.at[wid], idx_v)
        plsc.subcore_barrier()
        base = wid * _S_PER_W
        msgb = (msg0, msg1)

        def fire(g, buf):
            pltpu.async_copy(msg_hbm.at[pl.ds(base + g * _S_ROWS, _S_ROWS)],
                             msgb[buf], lsem.at[buf])

        fire(0, 0)

        @pl.loop(0, _S_NGRP, step=2)
        def _grp(g0):
            for p in range(2):
                g = g0 + p
                pltpu.make_async_copy(msg_hbm.at[pl.ds(0, _S_ROWS)],
                                      msgb[p], lsem.at[p]).wait()

                @pl.when(g + 1 < _S_NGRP)
                def _():
                    fire(g + 1, 1 - p)

                for q in range(_S_GRP):
                    pltpu.sync_copy(msgb[p].at[pl.ds(q * CH, CH)],
                                    acc_sh.at[idx_v.at[g * _S_GRP + q]],
                                    add=True)

        plsc.subcore_barrier()
        pltpu.sync_copy(acc_sh.at[pl.ds(sid * ROWS_PER_TILE, ROWS_PER_TILE)],
                        out_hbm.at[cid, pl.ds(sid * ROWS_PER_TILE, ROWS_PER_TILE)])

    return gather_k, scatter_k


def _sc_gather(table, gidx):
    return _sc_kernels()[0](table, gidx)


def _sc_scatter(msg, sidx, zeros_acc):
    return _sc_kernels()[1](msg, sidx, zeros_acc)


def kernel(x, edge_attr, params, edge_index):
    row = edge_index[0].astype(jnp.int32)
    col = edge_index[1].astype(jnp.int32)
    pad = E_PAD - E
    zero_idx = jnp.zeros((pad,), jnp.int32)
    gidx = jnp.concatenate([col, zero_idx, row, zero_idx]).reshape(NW, _G_NCH, CH)
    sidx = jnp.concatenate([col, jnp.full((pad,), N, jnp.int32)]).reshape(
        NW, _S_NCH, CH)
    zeros_acc = jnp.zeros((ACC_ROWS, LATENT), jnp.float32)

    node_lat = _run_enc(x, params["node_enc"], N, BLK_N, x.shape[1])
    ea_pad = jnp.pad(edge_attr, ((0, pad), (0, 0)))
    edge_lat = _run_enc(ea_pad, params["edge_enc"], E_PAD, BLK_E,
                        edge_attr.shape[1])

    nx = node_lat
    ne = edge_lat
    for _ in range(STEPS):
        gathered = _sc_gather(nx, gidx)
        msg, ne = _run_edge(gathered, ne, params["edge_net"])
        partials = _sc_scatter(msg, sidx, zeros_acc)
        nx = _run_node(partials, nx, params["node_net"])

    return _run_decode(nx, node_lat, params)
